# baseline (speedup 1.0000x reference)
import functools

import jax
import jax.numpy as jnp
from jax import lax
from jax.experimental import pallas as pl
from jax.experimental.pallas import tpu as pltpu
from jax.experimental.pallas import tpu_sc as plsc

_B = 16384
_D = 64
_V = 100000
_NBLK = 782
_NR = 50048


@functools.lru_cache(maxsize=None)
def _build():
    info = plsc.get_sparse_core_info()
    nc, ns = info.num_cores, info.num_subcores
    assert nc == 2 and ns == 16
    bpt = (_NBLK + ns - 1) // ns
    nj = _B // 128 // ns
    nt = 3 * nj
    mesh = plsc.VectorSubcoreMesh(core_axis_name="c", subcore_axis_name="s")
    out_type = (
        jax.ShapeDtypeStruct((6, 8, 128, 8, 128), jnp.float32),
        jax.ShapeDtypeStruct((_NR, 128), jnp.float32),
        jax.ShapeDtypeStruct((_NR, 128), jnp.float32),
    )

    @functools.partial(
        pl.kernel,
        mesh=mesh,
        out_type=out_type,
        compiler_params=pltpu.CompilerParams(
            use_tc_tiling_on_sc=True, needs_layout_passes=False),
        scratch_types=[
            pltpu.VMEM((3, nj, 128), jnp.int32),
            pltpu.VMEM((3, nj, 128), jnp.int32),
            pltpu.VMEM((_D, 128), jnp.float32),
            pltpu.VMEM((_D, 128), jnp.float32),
            pltpu.VMEM((_D, 128), jnp.float32),
            pltpu.VMEM((_D, 128), jnp.float32),
            pltpu.VMEM((128, 128), jnp.float32),
            pltpu.VMEM((128, 128), jnp.float32),
            pltpu.SemaphoreType.DMA,
            pltpu.SemaphoreType.DMA,
            pltpu.SemaphoreType.DMA,
            pltpu.SemaphoreType.DMA,
            pltpu.SemaphoreType.DMA,
            pltpu.SemaphoreType.DMA,
        ],
    )
    def gather6(pt_u, pt_t, tl_u, tl_t, i_u, i_tp, i_tn, i_up, i_un, i_ta,
                big, r_u, r_t, idx_v, pr_v, slab0, slab1, rblk0, rblk1,
                rows0, rows1, s_a0, s_a1, s_g0, s_g1, s_o0, s_o1):
        cid = lax.axis_index("c")
        sid = lax.axis_index("s")
        iota = lax.iota(jnp.int32, 16)
        slabs = (slab0, slab1)
        rblks = (rblk0, rblk1)
        rows = (rows0, rows1)
        sems_a = (s_a0, s_a1)
        sems_g = (s_g0, s_g1)
        sems_o = (s_o0, s_o1)
        rowvs = [iota + 16 * h for h in range(8)]

        per_core = (
            (pt_u, tl_u, r_u, (i_u, i_up, i_un),
             lambda gi: jnp.where(gi == 0, 0, gi + 2)),
            (pt_t, tl_t, r_t, (i_tp, i_tn, i_ta),
             lambda gi: jnp.where(gi == 2, 5, gi + 1)),
        )

        for tcore, (pt, tl, r_hbm, _irefs, _slotf) in enumerate(per_core):
            @pl.when(cid == tcore)
            def _phase_a(pt=pt, tl=tl, r_hbm=r_hbm):
                base = sid * bpt
                n = jnp.minimum(bpt, _NBLK - base)

                def issue_in(i, p):
                    c = base + i

                    @pl.when(i < n)
                    def _():
                        @pl.when(c < _NBLK - 1)
                        def _():
                            off = pl.multiple_of(128 * c, 128)
                            pltpu.async_copy(
                                pt.at[:, pl.ds(off, 128)], slabs[p],
                                sems_a[p])

                        @pl.when(c == _NBLK - 1)
                        def _():
                            pltpu.async_copy(tl, slabs[p], sems_a[p])

                def drain_in(i, p):
                    @pl.when(i < n)
                    def _():
                        pltpu.make_async_copy(
                            pt.at[:, pl.ds(0, 128)], slabs[p],
                            sems_a[p]).wait()

                def do_block(i, p):
                    c = base + i

                    @pl.when(i < n)
                    def _():
                        coff = jnp.where(c == _NBLK - 1, 96, 0)

                        def tp(o, _):
                            jv = (iota + o) & 63
                            for g in range(8):
                                colv = jnp.minimum(
                                    2 * jv + coff + (0 if g < 4 else 1),
                                    127)
                                v = plsc.load_gather(
                                    slabs[p], [rowvs[g % 4], colv])
                                plsc.store_scatter(
                                    rblks[p], [jv, iota + 16 * g], v)
                            return _
                        lax.fori_loop(0, 64, tp, 0)
                        off = pl.multiple_of(64 * c, 64)
                        pltpu.sync_copy(rblks[p],
                                        r_hbm.at[pl.ds(off, 64)])

                issue_in(0, 0)

                def body(q, _):
                    i0, i1 = 2 * q, 2 * q + 1
                    issue_in(i1, 1)
                    drain_in(i0, 0)
                    do_block(i0, 0)
                    issue_in(i0 + 2, 0)
                    drain_in(i1, 1)
                    do_block(i1, 1)
                    return _
                lax.fori_loop(0, (bpt + 1) // 2, body, 0)

        plsc.subcore_barrier()

        for tcore, (pt, tl, r_hbm, irefs, slotf) in enumerate(per_core):
            @pl.when(cid == tcore)
            def _phase_b(r_hbm=r_hbm, irefs=irefs, slotf=slotf):
                for gi in range(3):
                    pltpu.sync_copy(irefs[gi].at[pl.ds(sid * nj, nj)],
                                    idx_v.at[gi])

                    def mk_pr(cc, _, gi=gi):
                        for m in range(8):
                            v = idx_v[gi, cc, pl.ds(16 * m, 16)]
                            pr_v[gi, cc, pl.ds(16 * m, 16)] = v >> 1
                        return _
                    lax.fori_loop(0, nj, mk_pr, 0)

                def issue_g(t, p):
                    @pl.when(t < nt)
                    def _():
                        gi = t // nj
                        cc = lax.rem(t, nj)
                        pltpu.async_copy(
                            r_hbm.at[pr_v.at[gi, cc]], rows[p], sems_g[p])

                def drain_g(t, p):
                    @pl.when(t < nt)
                    def _():
                        pltpu.make_async_copy(
                            r_hbm.at[pl.ds(0, 128)], rows[p],
                            sems_g[p]).wait()

                def do_chunk(t, p):
                    @pl.when(t < nt)
                    def _():
                        gi = t // nj
                        cc = lax.rem(t, nj)
                        parvs = [
                            (idx_v[gi, cc, pl.ds(16 * h, 16)] & 1) << 6
                            for h in range(8)
                        ]

                        def tp(o, _):
                            kv = (iota + o) & 63
                            for h in range(8):
                                v = plsc.load_gather(
                                    rows[p], [rowvs[h], parvs[h] + kv])
                                plsc.store_scatter(
                                    rblks[p], [kv, rowvs[h]], v)
                            return _
                        lax.fori_loop(0, 64, tp, 0)
                        slot = slotf(gi)
                        c = sid * nj + cc
                        for g in range(8):
                            pltpu.sync_copy(
                                rblks[p].at[pl.ds(8 * g, 8)],
                                big.at[slot, g, c])

                issue_g(0, 0)

                def bodyb(q, _):
                    t0, t1 = 2 * q, 2 * q + 1
                    issue_g(t1, 1)
                    drain_g(t0, 0)
                    do_chunk(t0, 0)
                    issue_g(t0 + 2, 0)
                    drain_g(t1, 1)
                    do_chunk(t1, 1)
                    return _
                lax.fori_loop(0, nt // 2, bodyb, 0)

    return gather6


def kernel(x_user, x_track_pos, x_track_neg, x_user_pos, x_user_neg,
           x_track_anchor, users_vecs, tracks_vecs):
    gather6 = _build()

    def i2(x):
        return x.reshape(_B // 128, 128)

    big, _r1, _r2 = gather6(
        users_vecs.T, tracks_vecs.T,
        users_vecs[_V - 128:].T, tracks_vecs[_V - 128:].T,
        i2(x_user), i2(x_track_pos), i2(x_track_neg), i2(x_user_pos),
        i2(x_user_neg), i2(x_track_anchor))

    def fin(i):
        return big[i].transpose(1, 3, 0, 2).reshape(_B, _D)

    return tuple(fin(i) for i in range(6))


# speedup vs baseline: 1.3227x; 1.3227x over previous
import functools

import jax
import jax.numpy as jnp
from jax import lax
from jax.experimental import pallas as pl
from jax.experimental.pallas import tpu as pltpu
from jax.experimental.pallas import tpu_sc as plsc

_B = 16384
_D = 64
_V = 100000


@functools.lru_cache(maxsize=None)
def _build():
    info = plsc.get_sparse_core_info()
    nc, ns = info.num_cores, info.num_subcores
    nw = nc * ns
    nj = _B // 128 // nw
    nt = 3 * nj
    mesh = plsc.VectorSubcoreMesh(core_axis_name="c", subcore_axis_name="s")
    out_type = tuple(
        jax.ShapeDtypeStruct((8, 128, 8, 128), jnp.float32)
        for _ in range(3)
    )

    @functools.partial(
        pl.kernel,
        mesh=mesh,
        out_type=out_type,
        compiler_params=pltpu.CompilerParams(
            use_tc_tiling_on_sc=False, needs_layout_passes=False),
        scratch_types=[
            pltpu.VMEM((3, nj, 128), jnp.int32),
            pltpu.VMEM((128, 128), jnp.float32),
            pltpu.VMEM((128, 128), jnp.float32),
            pltpu.VMEM((_D, 128), jnp.float32),
            pltpu.VMEM((_D, 128), jnp.float32),
            pltpu.SemaphoreType.DMA,
            pltpu.SemaphoreType.DMA,
            pltpu.SemaphoreType.DMA,
        ],
    )
    def gather3(tbl, i_a, i_b, i_c, o_a, o_b, o_c,
                idx_v, rows0, rows1, outt0, outt1, sem_g, sem_o0, sem_o1):
        wid = lax.axis_index("s") * nc + lax.axis_index("c")
        iota = lax.iota(jnp.int32, 16)
        outs = (o_a, o_b, o_c)
        rows = (rows0, rows1)
        outts = (outt0, outt1)
        sems_o = (sem_o0, sem_o1)
        rowvs = [iota + 16 * h for h in range(8)]

        for gi, iref in enumerate((i_a, i_b, i_c)):
            pltpu.sync_copy(iref.at[pl.ds(wid * nj, nj)], idx_v.at[gi])

        chunks = [(gi, cc) for gi in range(3) for cc in range(nj)]
        g_hs = {}
        o_hs = {}
        g_hs[0] = pltpu.async_copy(tbl.at[idx_v.at[0, 0]], rows[0], sem_g)
        for t, (gi, cc) in enumerate(chunks):
            b = t & 1
            g_hs[t].wait()
            if t + 1 < nt:
                gi2, cc2 = chunks[t + 1]
                g_hs[t + 1] = pltpu.async_copy(
                    tbl.at[idx_v.at[gi2, cc2]], rows[1 - b], sem_g)
            if t >= 2:
                for h in o_hs.pop(t - 2):
                    h.wait()
            src, dst = rows[b], outts[b]

            def tp(ob, _, src=src, dst=dst):
                kv = (iota + ob) & 63
                for h in range(8):
                    v = plsc.load_gather(src, [rowvs[h], kv])
                    plsc.store_scatter(dst, [kv, rowvs[h]], v)
                return _
            lax.fori_loop(0, 64, tp, 0)
            c = wid * nj + cc
            o_hs[t] = [
                pltpu.async_copy(outts[b].at[pl.ds(8 * g, 8)],
                                 outs[gi].at[g, c], sems_o[b])
                for g in range(8)
            ]
        for t in (nt - 2, nt - 1):
            for h in o_hs.pop(t):
                h.wait()

    return gather3


def kernel(x_user, x_track_pos, x_track_neg, x_user_pos, x_user_neg,
           x_track_anchor, users_vecs, tracks_vecs):
    gather3 = _build()

    def i2(x):
        return x.reshape(_B // 128, 128)

    pd_u = jnp.pad(users_vecs, ((0, 0), (0, 64)))
    pd_t = jnp.pad(tracks_vecs, ((0, 0), (0, 64)))
    u4, up4, un4 = gather3(pd_u, i2(x_user), i2(x_user_pos), i2(x_user_neg))
    tp4, tn4, ta4 = gather3(pd_t, i2(x_track_pos), i2(x_track_neg),
                            i2(x_track_anchor))

    def fin(o):
        return o.transpose(1, 3, 0, 2).reshape(_B, _D)

    return (fin(u4), fin(tp4), fin(tn4), fin(up4), fin(un4), fin(ta4))
